# baseline (speedup 1.0000x reference)
import functools

import jax
import jax.numpy as jnp
from jax import lax
from jax.experimental import pallas as pl
from jax.experimental.pallas import tpu as pltpu, tpu_sc as plsc

_EMBED_DIM = 64
_BATCH = 16384


@jax.jit
def kernel(x, table):
    info = plsc.get_sparse_core_info()
    nw = info.num_cores * info.num_subcores
    b_per_w = _BATCH // nw
    mesh = plsc.VectorSubcoreMesh(core_axis_name="c", subcore_axis_name="s")

    @functools.partial(
        pl.kernel,
        mesh=mesh,
        out_type=jax.ShapeDtypeStruct((_BATCH, _EMBED_DIM), jnp.float32),
        scratch_types=[
            pltpu.VMEM((b_per_w,), jnp.int32),
            pltpu.VMEM((b_per_w, _EMBED_DIM), jnp.float32),
            pltpu.SemaphoreType.DMA,
        ],
        compiler_params=pltpu.CompilerParams(use_tc_tiling_on_sc=False),
    )
    def gather_kernel(x_hbm, table_hbm, out_hbm, idx_v, rows_v, sem):
        wid = lax.axis_index("s") * info.num_cores + lax.axis_index("c")
        base = wid * b_per_w
        pltpu.sync_copy(x_hbm.at[pl.ds(base, b_per_w)], idx_v)
        pltpu.async_copy(table_hbm.at[idx_v], rows_v, sem).wait()
        pltpu.sync_copy(rows_v, out_hbm.at[pl.ds(base, b_per_w)])

    return gather_kernel(x, table)


# speedup vs baseline: 1.6756x; 1.6756x over previous
import functools

import jax
import jax.numpy as jnp
from jax import lax
from jax.experimental import pallas as pl
from jax.experimental.pallas import tpu as pltpu, tpu_sc as plsc

_EMBED_DIM = 64
_BATCH = 16384
_CHUNK = 16


@jax.jit
def kernel(x, table):
    info = plsc.get_sparse_core_info()
    nw = info.num_cores * info.num_subcores
    b_per_w = _BATCH // nw
    n_chunks = b_per_w // _CHUNK
    mesh = plsc.VectorSubcoreMesh(core_axis_name="c", subcore_axis_name="s")

    @functools.partial(
        pl.kernel,
        mesh=mesh,
        out_type=jax.ShapeDtypeStruct((_BATCH, _EMBED_DIM), jnp.float32),
        scratch_types=[
            pltpu.VMEM((b_per_w,), jnp.int32),
            pltpu.VMEM((b_per_w, _EMBED_DIM), jnp.float32),
            pltpu.SemaphoreType.DMA,
        ],
    )
    def gather_kernel(x_hbm, table_hbm, out_hbm, idx_v, rows_v, sem):
        wid = lax.axis_index("s") * info.num_cores + lax.axis_index("c")
        base = wid * b_per_w
        pltpu.sync_copy(x_hbm.at[pl.ds(base, b_per_w)], idx_v)

        def fire(c):
            vec = idx_v[pl.ds(c * _CHUNK, _CHUNK)]
            for j in range(_CHUNK):
                pltpu.async_copy(
                    table_hbm.at[vec[j]], rows_v.at[c * _CHUNK + j], sem
                )

        def drain(c):
            pltpu.make_async_copy(
                table_hbm.at[pl.ds(0, _CHUNK)],
                rows_v.at[pl.ds(c * _CHUNK, _CHUNK)],
                sem,
            ).wait()

        fire(0)

        def body(c, _):
            fire(c + 1)
            drain(c)
            return 0

        lax.fori_loop(0, n_chunks - 1, body, 0)
        drain(n_chunks - 1)

        pltpu.sync_copy(rows_v, out_hbm.at[pl.ds(base, b_per_w)])

    return gather_kernel(x, table)


# speedup vs baseline: 2.3231x; 1.3864x over previous
import functools

import jax
import jax.numpy as jnp
from jax import lax
from jax.experimental import pallas as pl
from jax.experimental.pallas import tpu as pltpu, tpu_sc as plsc

_D = 64
_B = 16384
_C = 128
_NWIN = 7812
_SWEEP_ROWS = _NWIN * _C
_TAIL = 1000001 - _SWEEP_ROWS
_XCH = 4096


@jax.jit
def kernel(x, table):
    info = plsc.get_sparse_core_info()
    nw = info.num_cores * info.num_subcores
    wins_per_worker = 246
    mesh = plsc.VectorSubcoreMesh(core_axis_name="c", subcore_axis_name="s")

    table_t = jnp.swapaxes(table, 0, 1)
    tail_t = lax.slice(table, (_SWEEP_ROWS, 0), (1000001, _D))

    @functools.partial(
        pl.kernel,
        mesh=mesh,
        out_type=jax.ShapeDtypeStruct((_B, _D), jnp.float32),
        scratch_types=[
            pltpu.VMEM((_XCH,), jnp.int32),
            pltpu.VMEM((_B + 16,), jnp.int32),
            pltpu.VMEM((_B + 16,), jnp.int32),
            pltpu.VMEM((_B + 16,), jnp.int32),
            pltpu.VMEM((_B + 16,), jnp.int32),
            pltpu.VMEM((_D, _C), jnp.float32),
            pltpu.VMEM((_D, _C), jnp.float32),
            pltpu.VMEM((16, _D), jnp.float32),
            pltpu.SemaphoreType.DMA,
            pltpu.SemaphoreType.DMA,
            pltpu.SemaphoreType.DMA,
        ],
        compiler_params=pltpu.CompilerParams(needs_layout_passes=False),
    )
    def sweep_kernel(
        x_hbm, tt_hbm, tail_hbm, out_hbm,
        xch, mi, mp, hv, hp, wb0, wb1, rs, sem_w, sem_o, sem_m,
    ):
        w = lax.axis_index("s") * info.num_cores + lax.axis_index("c")
        lanes = lax.iota(jnp.int32, 16)

        def fire(j, buf):
            t = w + nw * j
            tf = jnp.where(t < _NWIN, t, 0)
            off = pl.multiple_of(tf * _C, _C)
            pltpu.async_copy(tt_hbm.at[:, pl.ds(off, _C)], buf, sem_w)

        def wait_win(buf):
            pltpu.make_async_copy(tt_hbm.at[:, pl.ds(0, _C)], buf, sem_w).wait()

        fire(0, wb0)

        def collect_chunk(c, cnt):
            pltpu.sync_copy(x_hbm.at[pl.ds(c * _XCH, _XCH)], xch)

            def inner(q, cnt):
                v = xch[pl.ds(q * 16, 16)]
                pos = c * _XCH + q * 16 + lanes
                m = ((v >> 7) & (nw - 1)) == w
                n = plsc.all_reduce_population_count(m)[0]
                plsc.store_compressed(mi.at[pl.ds(cnt, 16)], v, mask=m)
                plsc.store_compressed(mp.at[pl.ds(cnt, 16)], pos, mask=m)
                return cnt + n

            return lax.fori_loop(0, _XCH // 16, inner, cnt)

        count = lax.fori_loop(0, _B // _XCH, collect_chunk, 0)
        nq = (count + 15) // 16

        def process(t, buf):
            def scan(q, h):
                v = mi[pl.ds(q * 16, 16)]
                p = mp[pl.ds(q * 16, 16)]
                valid = (q * 16 + lanes) < count
                m = ((v >> 7) == t) & valid
                n = plsc.all_reduce_population_count(m)[0]
                plsc.store_compressed(hv.at[pl.ds(h, 16)], v, mask=m)
                plsc.store_compressed(hp.at[pl.ds(h, 16)], p, mask=m)
                return h + n

            h = lax.fori_loop(0, nq, scan, 0)

            def group(g, _):
                vv = hv[pl.ds(g * 16, 16)]
                pp = hp[pl.ds(g * 16, 16)]
                for j in range(16):
                    k = g * 16 + j

                    @pl.when(k < h)
                    def _hit():
                        vj = vv[j]
                        pj = pp[j]

                        @pl.when(t < _NWIN)
                        def _main():
                            lane = jnp.full((16,), vj & 127, jnp.int32)
                            for s in range(4):
                                rows = lanes + 16 * s
                                col = plsc.load_gather(buf, [rows, lane])
                                rs[j, pl.ds(16 * s, 16)] = col

                        @pl.when(t == _NWIN)
                        def _tail():
                            r = vj - _SWEEP_ROWS
                            pltpu.async_copy(tail_hbm.at[r], rs.at[j], sem_m)
                            pltpu.make_async_copy(
                                tail_hbm.at[0], rs.at[j], sem_m
                            ).wait()

                        pltpu.async_copy(rs.at[j], out_hbm.at[pj], sem_o)

                def drain(dj, _):
                    pltpu.make_async_copy(
                        out_hbm.at[pl.ds(0, 1)], rs.at[pl.ds(0, 1)], sem_o
                    ).wait()
                    return 0

                lax.fori_loop(0, jnp.minimum(h - g * 16, 16), drain, 0)
                return 0

            lax.fori_loop(0, (h + 15) // 16, group, 0)

        def sweep(i, _):
            j0 = 2 * i
            fire(j0 + 1, wb1)
            wait_win(wb0)
            process(w + nw * j0, wb0)
            fire(j0 + 2, wb0)
            wait_win(wb1)
            process(w + nw * (j0 + 1), wb1)
            return 0

        lax.fori_loop(0, wins_per_worker // 2, sweep, 0)
        wait_win(wb0)

    return sweep_kernel(x, table_t, tail_t)


# speedup vs baseline: 4.2110x; 1.8127x over previous
import functools

import jax
import jax.numpy as jnp
from jax import lax
from jax.experimental import pallas as pl
from jax.experimental.pallas import tpu as pltpu, tpu_sc as plsc

_D = 64
_B = 16384
_C = 512
_NWIN = 1953
_SWEEP_ROWS = _NWIN * _C
_XCH = 4096
_SENTINEL = 0x7FFF


@jax.jit
def kernel(x, table):
    info = plsc.get_sparse_core_info()
    nw = info.num_cores * info.num_subcores
    wins_per_worker = 62
    mesh = plsc.VectorSubcoreMesh(core_axis_name="c", subcore_axis_name="s")

    table_t = jnp.swapaxes(table, 0, 1)
    tail_t = lax.slice(table, (_SWEEP_ROWS, 0), (1000001, _D))

    @functools.partial(
        pl.kernel,
        mesh=mesh,
        out_type=jax.ShapeDtypeStruct((_B, _D), jnp.float32),
        scratch_types=[
            pltpu.VMEM((_XCH,), jnp.int32),
            pltpu.VMEM((_B + 16,), jnp.int32),
            pltpu.VMEM((_B + 16,), jnp.int32),
            pltpu.VMEM((_B + 16,), jnp.int32),
            pltpu.VMEM((_D, _C), jnp.float32),
            pltpu.VMEM((_D, _C), jnp.float32),
            pltpu.VMEM((16, _D), jnp.float32),
            pltpu.SemaphoreType.DMA,
            pltpu.SemaphoreType.DMA,
            pltpu.SemaphoreType.DMA,
        ],
        compiler_params=pltpu.CompilerParams(needs_layout_passes=False),
    )
    def sweep_kernel(
        x_hbm, tt_hbm, tail_hbm, out_hbm,
        xch, mw, mpk, hu, wb0, wb1, rs, sem_w, sem_o, sem_m,
    ):
        w = lax.axis_index("s") * info.num_cores + lax.axis_index("c")
        lanes = lax.iota(jnp.int32, 16)

        def fire(j, buf):
            t = w + nw * j
            tf = jnp.where(t < _NWIN, t, 0)
            off = pl.multiple_of(tf * _C, _C)
            pltpu.async_copy(tt_hbm.at[:, pl.ds(off, _C)], buf, sem_w)

        def wait_win(buf):
            pltpu.make_async_copy(tt_hbm.at[:, pl.ds(0, _C)], buf, sem_w).wait()

        fire(0, wb0)

        def collect_chunk(c, cnt):
            pltpu.sync_copy(x_hbm.at[pl.ds(c * _XCH, _XCH)], xch)

            def inner(q, cnt):
                v = xch[pl.ds(q * 16, 16)]
                win = v >> 9
                pos = c * _XCH + q * 16 + lanes
                m = (win & (nw - 1)) == w
                n = plsc.all_reduce_population_count(m)[0]
                plsc.store_compressed(mw.at[pl.ds(cnt, 16)], win, mask=m)
                packed = (pos << 9) | (v & (_C - 1))
                plsc.store_compressed(mpk.at[pl.ds(cnt, 16)], packed, mask=m)
                return cnt + n

            return lax.fori_loop(0, _XCH // 16, inner, cnt)

        count = lax.fori_loop(0, _B // _XCH, collect_chunk, 0)
        mw[pl.ds(count, 16)] = jnp.full((16,), _SENTINEL, jnp.int32)
        nq = (count + 15) // 16

        def process(t, buf):
            def scan(q, h):
                wv = mw[pl.ds(q * 16, 16)]
                m = wv == t
                n = plsc.all_reduce_population_count(m)[0]
                plsc.store_compressed(
                    hu.at[pl.ds(h, 16)], mpk[pl.ds(q * 16, 16)], mask=m
                )
                return h + n

            h = lax.fori_loop(0, nq, scan, 0)

            def group(g, _):
                uu = hu[pl.ds(g * 16, 16)]
                for j in range(16):
                    k = g * 16 + j

                    @pl.when(k < h)
                    def _hit():
                        uj = uu[j]
                        lane = uj & (_C - 1)
                        pj = uj >> 9

                        @pl.when(t < _NWIN)
                        def _main():
                            lanev = jnp.full((16,), lane, jnp.int32)
                            for s in range(4):
                                rows = lanes + 16 * s
                                col = plsc.load_gather(buf, [rows, lanev])
                                rs[j, pl.ds(16 * s, 16)] = col

                        @pl.when(t == _NWIN)
                        def _tail():
                            pltpu.async_copy(tail_hbm.at[lane], rs.at[j], sem_m)
                            pltpu.make_async_copy(
                                tail_hbm.at[0], rs.at[j], sem_m
                            ).wait()

                        pltpu.async_copy(rs.at[j], out_hbm.at[pj], sem_o)

                def drain(dj, _):
                    pltpu.make_async_copy(
                        out_hbm.at[pl.ds(0, 1)], rs.at[pl.ds(0, 1)], sem_o
                    ).wait()
                    return 0

                lax.fori_loop(0, jnp.minimum(h - g * 16, 16), drain, 0)
                return 0

            lax.fori_loop(0, (h + 15) // 16, group, 0)

        def sweep(i, _):
            j0 = 2 * i
            fire(j0 + 1, wb1)
            wait_win(wb0)
            process(w + nw * j0, wb0)
            fire(j0 + 2, wb0)
            wait_win(wb1)
            process(w + nw * (j0 + 1), wb1)
            return 0

        lax.fori_loop(0, wins_per_worker // 2, sweep, 0)
        wait_win(wb0)

    return sweep_kernel(x, table_t, tail_t)
